# Initial kernel scaffold; baseline (speedup 1.0000x reference)
import jax
import jax.numpy as jnp
from jax.experimental import pallas as pl


def kernel(features, id_embedding, edge_index, preference, W_mlp, b_mlp, conv1_w, lin1_w, lin1_b, g1_w, g1_b, conv2_w, lin2_w, lin2_b, g2_w, g2_b, conv3_w, lin3_w, lin3_b, g3_w, g3_b):
    raise NotImplementedError("write your pallas kernel here")


# speedup vs baseline: 3.7970x; 3.7970x over previous
import functools

import jax
import jax.numpy as jnp
from jax import lax
from jax.experimental import pallas as pl
from jax.experimental.pallas import tpu as pltpu
from jax.experimental.pallas import tpu_sc as plsc

N = 10000
D = 64
W128 = 128
E = 320000
NC = 2
NS = 16
NW = NC * NS
CS = 128
CH = 79
EPAD = NW * CH * CS
R = 10112
RT = R // NS

_mesh = plsc.VectorSubcoreMesh(core_axis_name="c", subcore_axis_name="s",
                               num_cores=NC, num_subcores=NS)


@functools.partial(
    pl.kernel,
    out_type=jax.ShapeDtypeStruct((NC, R, W128), jnp.float32),
    mesh=_mesh,
    scratch_types=[
        pltpu.VMEM((CH, CS), jnp.int32),
        pltpu.VMEM((CH, CS), jnp.int32),
        pltpu.VMEM((CS, W128), jnp.float32),
        pltpu.VMEM_SHARED((R, W128), jnp.float32),
        pltpu.SemaphoreType.DMA,
    ],
)
def _sc_agg(table_hbm, src_hbm, dst_hbm, z_hbm, out_hbm,
            src_v, dst_v, rows_v, acc, sem):
    c = lax.axis_index("c")
    s = lax.axis_index("s")
    wid = c * NS + s
    pltpu.sync_copy(z_hbm.at[pl.ds(s * RT, RT)], acc.at[pl.ds(s * RT, RT)])
    pltpu.sync_copy(src_hbm.at[wid], src_v)
    pltpu.sync_copy(dst_hbm.at[wid], dst_v)
    plsc.subcore_barrier()

    def body(j, carry):
        pltpu.async_copy(table_hbm.at[src_v.at[j]], rows_v, sem).wait()
        pltpu.sync_copy(rows_v, acc.at[dst_v.at[j]], add=True)
        return carry

    lax.fori_loop(0, CH, body, 0)
    plsc.subcore_barrier()
    pltpu.sync_copy(acc.at[pl.ds(s * RT, RT)],
                    out_hbm.at[c, pl.ds(s * RT, RT)])


def _lrelu(v):
    return jnp.where(v >= 0, v, 0.01 * v)


def _mlp_body(f_ref, w_ref, b_ref, o_ref):
    o_ref[...] = (jnp.dot(f_ref[...], w_ref[...],
                          preferred_element_type=jnp.float32) + b_ref[...])


def _norm_body(x_ref, o_ref):
    x = x_ref[...]
    n = jnp.sqrt(jnp.sum(x * x, axis=1, keepdims=True))
    o_ref[...] = x / jnp.maximum(n, 1e-12)


def _layer_body(x_ref, p0_ref, p1_ref, c0_ref, c1_ref, id_ref,
                cw_ref, lw_ref, lb_ref, gw_ref, gb_ref, o_ref):
    cnt = jnp.maximum(c0_ref[...] + c1_ref[...], 1.0)[:, 0:1]
    sagg = (p0_ref[...] + p1_ref[...]) / cnt
    h = _lrelu(jnp.dot(sagg, cw_ref[...], preferred_element_type=jnp.float32))
    x_hat = _lrelu(jnp.dot(x_ref[...], lw_ref[...],
                           preferred_element_type=jnp.float32)
                   + lb_ref[...]) + id_ref[...]
    o_ref[...] = _lrelu(jnp.dot(h, gw_ref[...],
                                preferred_element_type=jnp.float32)
                        + gb_ref[...] + x_hat)


_BR = 1000


def _tc_mlp(features, w, b):
    m = features.shape[0]
    return pl.pallas_call(
        _mlp_body,
        grid=(m // _BR,),
        in_specs=[
            pl.BlockSpec((_BR, 128), lambda i: (i, 0)),
            pl.BlockSpec((128, D), lambda i: (0, 0)),
            pl.BlockSpec((1, D), lambda i: (0, 0)),
        ],
        out_specs=pl.BlockSpec((_BR, D), lambda i: (i, 0)),
        out_shape=jax.ShapeDtypeStruct((m, D), jnp.float32),
    )(features, w, b.reshape(1, D))


def _tc_norm(x):
    return pl.pallas_call(
        _norm_body,
        grid=(N // _BR,),
        in_specs=[pl.BlockSpec((_BR, D), lambda i: (i, 0))],
        out_specs=pl.BlockSpec((_BR, D), lambda i: (i, 0)),
        out_shape=jax.ShapeDtypeStruct((N, D), jnp.float32),
    )(x)


def _tc_layer(x, p0, p1, c0, c1, id_emb, cw, lw, lb, gw, gb):
    row = pl.BlockSpec((_BR, D), lambda i: (i, 0))
    mat = pl.BlockSpec((D, D), lambda i: (0, 0))
    vec = pl.BlockSpec((1, D), lambda i: (0, 0))
    return pl.pallas_call(
        _layer_body,
        grid=(N // _BR,),
        in_specs=[row, row, row,
                  pl.BlockSpec((_BR, 16), lambda i: (i, 0)),
                  pl.BlockSpec((_BR, 16), lambda i: (i, 0)),
                  row, mat, mat, vec, mat, vec],
        out_specs=row,
        out_shape=jax.ShapeDtypeStruct((N, D), jnp.float32),
    )(x, p0, p1, c0, c1, id_emb, cw, lw, lb.reshape(1, D), gw,
      gb.reshape(1, D))


def kernel(features, id_embedding, edge_index, preference, W_mlp, b_mlp,
           conv1_w, lin1_w, lin1_b, g1_w, g1_b,
           conv2_w, lin2_w, lin2_b, g2_w, g2_b,
           conv3_w, lin3_w, lin3_b, g3_w, g3_b):
    src = edge_index[0]
    dst = edge_index[1]
    pad = EPAD - E
    srcp = jnp.concatenate([src, jnp.zeros((pad,), jnp.int32)]
                           ).reshape(NW, CH, CS)
    dstp = jnp.concatenate([dst, jnp.full((pad,), N, jnp.int32)]
                           ).reshape(NW, CH, CS)
    zeros128 = jnp.zeros((R, W128), jnp.float32)
    onescol = jnp.concatenate(
        [jnp.ones((N, 1), jnp.float32), jnp.zeros((R - N, 1), jnp.float32)])
    tailcols = jnp.zeros((R, W128 - D - 1), jnp.float32)
    padrows = jnp.zeros((R - N, D), jnp.float32)

    temp = _tc_mlp(features, W_mlp, b_mlp)
    x = jnp.concatenate([preference, temp], axis=0)
    x = _tc_norm(x)

    c0 = c1 = None
    for cw, lw, lb, gw, gb in (
        (conv1_w, lin1_w, lin1_b, g1_w, g1_b),
        (conv2_w, lin2_w, lin2_b, g2_w, g2_b),
        (conv3_w, lin3_w, lin3_b, g3_w, g3_b),
    ):
        table = jnp.concatenate(
            [jnp.concatenate([x, padrows]), onescol, tailcols], axis=1)
        p = _sc_agg(table, srcp, dstp, zeros128)
        if c0 is None:
            c0 = p[0, :N, D:D + 16]
            c1 = p[1, :N, D:D + 16]
        x = _tc_layer(x, p[0, :N, :D], p[1, :N, :D], c0, c1, id_embedding,
                      cw, lw, lb, gw, gb)
    return x
